# baseline (device time: 12294 ns/iter reference)
import jax
import jax.numpy as jnp
from jax import lax
from jax.experimental import pallas as pl
from jax.experimental.pallas import tpu as pltpu

C = 4


def kernel(x):
    m, n = x.shape
    half = m // 2
    ch = half // C

    def body(x_ref, out_ref, send_ref, xrecv_ref, fwd_ref, zrecv_ref,
             xs_sems, xr_sems, zs_sems, zr_sems):
        my_x = lax.axis_index("x")
        my_y = lax.axis_index("y")
        my_z = lax.axis_index("z")
        xpeer = (1 - my_x, my_y, my_z)
        zpeer = (my_x, my_y, 1 - my_z)
        base = my_z * half
        other = (1 - my_z) * half

        send_ref[...] = x_ref[pl.ds(base, half), :].astype(jnp.bfloat16)

        barrier_sem = pltpu.get_barrier_semaphore()
        for nbr in (xpeer, zpeer):
            pl.semaphore_signal(
                barrier_sem, inc=1, device_id=nbr,
                device_id_type=pl.DeviceIdType.MESH,
            )
        pl.semaphore_wait(barrier_sem, 2)

        x_rdmas = []
        for c in range(C):
            sl = pl.ds(c * ch, ch)
            r = pltpu.make_async_remote_copy(
                src_ref=send_ref.at[sl],
                dst_ref=xrecv_ref.at[sl],
                send_sem=xs_sems.at[c],
                recv_sem=xr_sems.at[c],
                device_id=xpeer,
                device_id_type=pl.DeviceIdType.MESH,
            )
            r.start()
            x_rdmas.append(r)

        z_rdmas = []
        for c in range(C):
            sl = pl.ds(c * ch, ch)
            x_rdmas[c].wait_recv()
            red = (x_ref[pl.ds(base + c * ch, ch), :]
                   + xrecv_ref[sl, :].astype(jnp.float32))
            out_ref[pl.ds(base + c * ch, ch), :] = red
            fwd_ref[sl, :] = red.astype(jnp.bfloat16)
            r = pltpu.make_async_remote_copy(
                src_ref=fwd_ref.at[sl],
                dst_ref=zrecv_ref.at[sl],
                send_sem=zs_sems.at[c],
                recv_sem=zr_sems.at[c],
                device_id=zpeer,
                device_id_type=pl.DeviceIdType.MESH,
            )
            r.start()
            z_rdmas.append(r)

        for c in range(C):
            sl = pl.ds(c * ch, ch)
            z_rdmas[c].wait_recv()
            out_ref[pl.ds(other + c * ch, ch), :] = (
                zrecv_ref[sl, :].astype(jnp.float32))

        for c in range(C):
            x_rdmas[c].wait_send()
            z_rdmas[c].wait_send()

    return pl.pallas_call(
        body,
        out_shape=jax.ShapeDtypeStruct((m, n), jnp.float32),
        in_specs=[pl.BlockSpec(memory_space=pltpu.VMEM)],
        out_specs=pl.BlockSpec(memory_space=pltpu.VMEM),
        scratch_shapes=[
            pltpu.VMEM((half, n), jnp.bfloat16),
            pltpu.VMEM((half, n), jnp.bfloat16),
            pltpu.VMEM((half, n), jnp.bfloat16),
            pltpu.VMEM((half, n), jnp.bfloat16),
            pltpu.SemaphoreType.DMA((C,)),
            pltpu.SemaphoreType.DMA((C,)),
            pltpu.SemaphoreType.DMA((C,)),
            pltpu.SemaphoreType.DMA((C,)),
        ],
        compiler_params=pltpu.CompilerParams(collective_id=0),
    )(x)
